# baseline (speedup 1.0000x reference)
import functools

import jax
import jax.numpy as jnp
from jax import lax
from jax.experimental import pallas as pl
from jax.experimental.pallas import tpu as pltpu
from jax.experimental.pallas import tpu_sc as plsc

F = 32
BATCH = 16384

NC = 2
NS = 16
NW = NC * NS
BPW = BATCH // NW
CHUNK = 128
NCHUNK = BPW // CHUNK
L = 16


def _body(user_hbm, item_hbm, uft_hbm, ift_hbm, out_hbm,
          uidx, iidx, ut, vt, oloc, sem):
    wid = lax.axis_index("s") * NC + lax.axis_index("c")
    base = wid * BPW

    for j in range(NCHUNK):
        pltpu.sync_copy(user_hbm.at[pl.ds(base + j * CHUNK, CHUNK)], uidx.at[j])
        pltpu.sync_copy(item_hbm.at[pl.ds(base + j * CHUNK, CHUNK)], iidx.at[j])

    def fire(f, _):
        for j in range(NCHUNK):
            pltpu.async_copy(uft_hbm.at[f].at[uidx.at[j]],
                             ut.at[f, pl.ds(j * CHUNK, CHUNK)], sem)
            pltpu.async_copy(ift_hbm.at[f].at[iidx.at[j]],
                             vt.at[f, pl.ds(j * CHUNK, CHUNK)], sem)
        return 0

    lax.fori_loop(0, F, fire, 0)
    pltpu.make_async_copy(
        uft_hbm.at[pl.ds(0, F), pl.ds(0, BPW)], ut, sem).wait()
    pltpu.make_async_copy(
        ift_hbm.at[pl.ds(0, F), pl.ds(0, BPW)], vt, sem).wait()

    def reduce_group(g, _):
        b0 = g * L
        acc = ut[0, pl.ds(b0, L)] * vt[0, pl.ds(b0, L)]
        for f in range(1, F):
            acc = acc + ut[f, pl.ds(b0, L)] * vt[f, pl.ds(b0, L)]
        oloc[pl.ds(b0, L)] = acc
        return 0

    lax.fori_loop(0, BPW // L, reduce_group, 0)

    pltpu.sync_copy(oloc, out_hbm.at[pl.ds(base, BPW)])


@jax.jit
def _mf_scores(user, item, user_factors, item_factors):
    mesh = plsc.VectorSubcoreMesh(core_axis_name="c", subcore_axis_name="s")
    kfn = functools.partial(
        pl.kernel,
        out_type=jax.ShapeDtypeStruct((BATCH,), jnp.float32),
        mesh=mesh,
        compiler_params=pltpu.CompilerParams(
            needs_layout_passes=False, use_tc_tiling_on_sc=False),
        scratch_types=[
            pltpu.VMEM((NCHUNK, CHUNK), jnp.int32),
            pltpu.VMEM((NCHUNK, CHUNK), jnp.int32),
            pltpu.VMEM((F, BPW), jnp.float32),
            pltpu.VMEM((F, BPW), jnp.float32),
            pltpu.VMEM((BPW,), jnp.float32),
            pltpu.SemaphoreType.DMA,
        ],
    )(_body)
    return kfn(user, item, user_factors.T, item_factors.T)


def kernel(user, item, user_factors, item_factors):
    return _mf_scores(user.astype(jnp.int32), item.astype(jnp.int32),
                      user_factors, item_factors)
